# baseline (speedup 1.0000x reference)
import functools

import jax
import jax.numpy as jnp
from jax import lax
from jax.experimental import pallas as pl
from jax.experimental.pallas import tpu as pltpu
from jax.experimental.pallas import tpu_sc as plsc

B = 16384
D = 64
NC = 2
NS = 16
L = 16
NW = NC * NS
BPW = B // NW
CHUNK = 128
NCHUNK = BPW // CHUNK
GROUPS = CHUNK // L


def _score_body(h2, t2, r2, enth, entt, rel, out, idx_h, idx_t, idx_r,
                hrows, trows, rrows, outv, semh, semt, semr):
    wid = lax.axis_index("s") * NC + lax.axis_index("c")
    base_row = wid * NCHUNK
    pltpu.sync_copy(h2.at[pl.ds(base_row, NCHUNK)], idx_h)
    pltpu.sync_copy(t2.at[pl.ds(base_row, NCHUNK)], idx_t)
    pltpu.sync_copy(r2.at[pl.ds(base_row, NCHUNK)], idx_r)
    lanes = lax.iota(jnp.int32, L)

    for c in range(NCHUNK):
        ch = pltpu.async_copy(enth.at[idx_h.at[c]], hrows, semh)
        ct = pltpu.async_copy(entt.at[idx_t.at[c]], trows, semt)
        cr = pltpu.async_copy(rel.at[idx_r.at[c]], rrows, semr)
        ch.wait()
        ct.wait()
        cr.wait()

        def group(g, carry, c=c):
            sums = jnp.zeros((L,), jnp.float32)
            for j in range(L):
                row = g * L + j
                s = None
                for q in range(D // L):
                    eh = hrows[row, pl.ds(q * L, L)]
                    er = rrows[row, pl.ds(q * L, L)]
                    et = plsc.bitcast(trows[row, pl.ds(q * L, L)],
                                      jnp.float32)
                    d = (eh - et) + er
                    s = d * d if s is None else s + d * d
                tot = jnp.sum(s)
                sums = jnp.where(lanes == j, tot, sums)
            x = sums + 1e-12
            i = plsc.bitcast(x, jnp.int32)
            i = 0x5F3759DF - lax.shift_right_logical(i, 1)
            y = plsc.bitcast(i, jnp.float32)
            for _ in range(3):
                y = y * (1.5 - 0.5 * x * y * y)
            outv[pl.ds(c * CHUNK + g * L, L)] = -(x * y)
            return carry

        lax.fori_loop(0, GROUPS, group, 0)

    pltpu.sync_copy(outv, out.at[pl.ds(wid * BPW, BPW)])


@jax.jit
def kernel(h, t, r, entity_emb, relation_emb):
    h2 = h.astype(jnp.int32).reshape(B // CHUNK, CHUNK)
    t2 = t.astype(jnp.int32).reshape(B // CHUNK, CHUNK)
    r2 = r.astype(jnp.int32).reshape(B // CHUNK, CHUNK)
    mesh = plsc.VectorSubcoreMesh(
        core_axis_name="c", subcore_axis_name="s",
        num_cores=NC, num_subcores=NS)
    run = pl.kernel(
        _score_body,
        out_type=jax.ShapeDtypeStruct((B,), jnp.float32),
        mesh=mesh,
        compiler_params=pltpu.CompilerParams(
            needs_layout_passes=False, use_tc_tiling_on_sc=False),
        scratch_types=[
            pltpu.VMEM((NCHUNK, CHUNK), jnp.int32),
            pltpu.VMEM((NCHUNK, CHUNK), jnp.int32),
            pltpu.VMEM((NCHUNK, CHUNK), jnp.int32),
            pltpu.VMEM((CHUNK, D), jnp.float32),
            pltpu.VMEM((CHUNK, D), jnp.uint32),
            pltpu.VMEM((CHUNK, D), jnp.float32),
            pltpu.VMEM((BPW,), jnp.float32),
            pltpu.SemaphoreType.DMA,
            pltpu.SemaphoreType.DMA,
            pltpu.SemaphoreType.DMA,
        ],
    )
    ent_u = lax.bitcast_convert_type(entity_emb, jnp.uint32)
    return run(h2, t2, r2, entity_emb, ent_u, relation_emb)


# speedup vs baseline: 1.9500x; 1.9500x over previous
import functools

import jax
import jax.numpy as jnp
from jax import lax
from jax.experimental import pallas as pl
from jax.experimental.pallas import tpu as pltpu
from jax.experimental.pallas import tpu_sc as plsc

B = 16384
D = 64
NC = 1
NS = 16
L = 16
NW = NC * NS
BPW = B // NW
CHUNK = 128
NCHUNK = BPW // CHUNK
GROUPS = CHUNK // L


def _score_body(h2, t2, r2, enth, entt, rel, out, idx_h, idx_t, idx_r,
                hrows, trows, rrows, outv, semh, semt, semr):
    wid = lax.axis_index("s") * NC + lax.axis_index("c")
    base_row = wid * NCHUNK
    pltpu.sync_copy(h2.at[pl.ds(base_row, NCHUNK)], idx_h)
    pltpu.sync_copy(t2.at[pl.ds(base_row, NCHUNK)], idx_t)
    pltpu.sync_copy(r2.at[pl.ds(base_row, NCHUNK)], idx_r)
    lanes = lax.iota(jnp.int32, L)

    for c in range(NCHUNK):
        ch = pltpu.async_copy(enth.at[idx_h.at[c]], hrows, semh)
        ct = pltpu.async_copy(entt.at[idx_t.at[c]], trows, semt)
        cr = pltpu.async_copy(rel.at[idx_r.at[c]], rrows, semr)
        ch.wait()
        ct.wait()
        cr.wait()

        def group(g, carry, c=c):
            sums = jnp.zeros((L,), jnp.float32)
            for j in range(L):
                row = g * L + j
                s = None
                for q in range(D // L):
                    eh = hrows[row, pl.ds(q * L, L)]
                    er = rrows[row, pl.ds(q * L, L)]
                    et = trows[row, pl.ds(q * L, L)]
                    d = (eh - et) + er
                    s = d * d if s is None else s + d * d
                tot = jnp.sum(s)
                sums = jnp.where(lanes == j, tot, sums)
            x = sums + 1e-12
            i = plsc.bitcast(x, jnp.int32)
            i = 0x5F3759DF - lax.shift_right_logical(i, 1)
            y = plsc.bitcast(i, jnp.float32)
            for _ in range(3):
                y = y * (1.5 - 0.5 * x * y * y)
            outv[pl.ds(c * CHUNK + g * L, L)] = -(x * y)
            return carry

        lax.fori_loop(0, GROUPS, group, 0)

    pltpu.sync_copy(outv, out.at[pl.ds(wid * BPW, BPW)])


@jax.jit
def kernel(h, t, r, entity_emb, relation_emb):
    h2 = h.astype(jnp.int32).reshape(B // CHUNK, CHUNK)
    t2 = t.astype(jnp.int32).reshape(B // CHUNK, CHUNK)
    r2 = r.astype(jnp.int32).reshape(B // CHUNK, CHUNK)
    mesh = plsc.VectorSubcoreMesh(
        core_axis_name="c", subcore_axis_name="s",
        num_cores=NC, num_subcores=NS)
    run = pl.kernel(
        _score_body,
        out_type=jax.ShapeDtypeStruct((B,), jnp.float32),
        mesh=mesh,
        compiler_params=pltpu.CompilerParams(
            needs_layout_passes=False, use_tc_tiling_on_sc=False),
        scratch_types=[
            pltpu.VMEM((NCHUNK, CHUNK), jnp.int32),
            pltpu.VMEM((NCHUNK, CHUNK), jnp.int32),
            pltpu.VMEM((NCHUNK, CHUNK), jnp.int32),
            pltpu.VMEM((CHUNK, D), jnp.float32),
            pltpu.VMEM((CHUNK, D), jnp.float32),
            pltpu.VMEM((CHUNK, D), jnp.float32),
            pltpu.VMEM((BPW,), jnp.float32),
            pltpu.SemaphoreType.DMA,
            pltpu.SemaphoreType.DMA,
            pltpu.SemaphoreType.DMA,
        ],
    )
    return run(h2, t2, r2, entity_emb, entity_emb, relation_emb)


# speedup vs baseline: 3.2261x; 1.6544x over previous
import functools

import jax
import jax.numpy as jnp
from jax import lax
from jax.experimental import pallas as pl
from jax.experimental.pallas import tpu as pltpu
from jax.experimental.pallas import tpu_sc as plsc

B = 16384
D = 64
NREL = 1000
NC = 2
NS = 16
L = 16
NW = NC * NS
BPW = B // NW
IDXW = 128
NIDX = BPW // IDXW
NBATCH = BPW // L


def _score_body(h1, t1, r1, ent, rel2, out, oid_h, oid_t, oid_r, relv,
                bufh0, bufh1, bufh2, bufh3, buft0, buft1, buft2, buft3, outv,
                semh0, semh1, semh2, semh3, semt0, semt1, semt2, semt3):
    wid = lax.axis_index("s") * NC + lax.axis_index("c")
    base = wid * BPW
    for b in range(NIDX):
        pltpu.sync_copy(h1.at[pl.ds(base + b * IDXW, IDXW)], oid_h.at[b])
        pltpu.sync_copy(t1.at[pl.ds(base + b * IDXW, IDXW)], oid_t.at[b])
        pltpu.sync_copy(r1.at[pl.ds(base + b * IDXW, IDXW)], oid_r.at[b])
    pltpu.sync_copy(rel2, relv)
    lanes = lax.iota(jnp.int32, L)

    bufh = (bufh0, bufh1, bufh2, bufh3)
    buft = (buft0, buft1, buft2, buft3)
    semh = (semh0, semh1, semh2, semh3)
    semt = (semt0, semt1, semt2, semt3)

    def extract(v, j):
        return jnp.sum(jnp.where(lanes == j, v, 0))

    def idx_vecs(bi, ref):
        b = bi // (IDXW // L)
        off = (bi % (IDXW // L)) * L
        return ref[b, pl.ds(off, L)]

    def issue(bi, s):
        vh = idx_vecs(bi, oid_h)
        vt = idx_vecs(bi, oid_t)
        for j in range(L):
            hj = extract(vh, j)
            tj = extract(vt, j)
            pltpu.async_copy(
                ent.at[pl.ds(hj, 1)], bufh[s].at[pl.ds(j, 1)], semh[s])
            pltpu.async_copy(
                ent.at[pl.ds(tj, 1)], buft[s].at[pl.ds(j, 1)], semt[s])

    def drain(s):
        pltpu.make_async_copy(ent.at[pl.ds(0, L)], bufh[s], semh[s]).wait()
        pltpu.make_async_copy(ent.at[pl.ds(0, L)], buft[s], semt[s]).wait()

    def compute(bi, s):
        vr = idx_vecs(bi, oid_r)
        sums = jnp.zeros((L,), jnp.float32)
        for j in range(L):
            rj = extract(vr, j)
            m = lax.shift_right_logical(rj, 1)
            o = jnp.bitwise_and(rj, 1) * D
            acc = None
            for q in range(D // L):
                eh = bufh[s][j, pl.ds(q * L, L)]
                et = buft[s][j, pl.ds(q * L, L)]
                er = relv[m, pl.ds(o + q * L, L)]
                d = (eh - et) + er
                acc = d * d if acc is None else acc + d * d
            tot = jnp.sum(acc)
            sums = jnp.where(lanes == j, tot, sums)
        x = sums + 1e-12
        ib = plsc.bitcast(x, jnp.int32)
        ib = 0x5F3759DF - lax.shift_right_logical(ib, 1)
        y = plsc.bitcast(ib, jnp.float32)
        for _ in range(3):
            y = y * (1.5 - 0.5 * x * y * y)
        outv[pl.ds(bi * L, L)] = -(x * y)

    for s in range(4):
        issue(s, s)

    def body(i, carry):
        for s in range(4):
            drain(s)
            compute(4 * i + s, s)

            @pl.when(4 * i + s + 4 < NBATCH)
            def _(s=s):
                issue(4 * i + s + 4, s)

        return carry

    lax.fori_loop(0, NBATCH // 4, body, 0)
    pltpu.sync_copy(outv, out.at[pl.ds(base, BPW)])


@jax.jit
def kernel(h, t, r, entity_emb, relation_emb):
    h1 = h.astype(jnp.int32)
    t1 = t.astype(jnp.int32)
    r1 = r.astype(jnp.int32)
    rel2 = relation_emb.reshape(NREL // 2, 2 * D)
    mesh = plsc.VectorSubcoreMesh(
        core_axis_name="c", subcore_axis_name="s",
        num_cores=NC, num_subcores=NS)
    run = pl.kernel(
        _score_body,
        out_type=jax.ShapeDtypeStruct((B,), jnp.float32),
        mesh=mesh,
        compiler_params=pltpu.CompilerParams(needs_layout_passes=False),
        scratch_types=[
            pltpu.VMEM((NIDX, IDXW), jnp.int32),
            pltpu.VMEM((NIDX, IDXW), jnp.int32),
            pltpu.VMEM((NIDX, IDXW), jnp.int32),
            pltpu.VMEM((NREL // 2, 2 * D), jnp.float32),
            pltpu.VMEM((L, D), jnp.float32),
            pltpu.VMEM((L, D), jnp.float32),
            pltpu.VMEM((L, D), jnp.float32),
            pltpu.VMEM((L, D), jnp.float32),
            pltpu.VMEM((L, D), jnp.float32),
            pltpu.VMEM((L, D), jnp.float32),
            pltpu.VMEM((L, D), jnp.float32),
            pltpu.VMEM((L, D), jnp.float32),
            pltpu.VMEM((BPW,), jnp.float32),
            pltpu.SemaphoreType.DMA,
            pltpu.SemaphoreType.DMA,
            pltpu.SemaphoreType.DMA,
            pltpu.SemaphoreType.DMA,
            pltpu.SemaphoreType.DMA,
            pltpu.SemaphoreType.DMA,
            pltpu.SemaphoreType.DMA,
            pltpu.SemaphoreType.DMA,
        ],
    )
    return run(h1, t1, r1, entity_emb, rel2)


# speedup vs baseline: 3.3059x; 1.0247x over previous
import functools

import jax
import jax.numpy as jnp
from jax import lax
from jax.experimental import pallas as pl
from jax.experimental.pallas import tpu as pltpu
from jax.experimental.pallas import tpu_sc as plsc

B = 16384
D = 64
NREL = 1000
NC = 2
NS = 16
L = 16
NW = NC * NS
BPW = B // NW
IDXW = 128
NIDX = BPW // IDXW
NBATCH = BPW // L


def _score_body(h1, t1, r1, ent, rel2, out, oid_h, oid_t, oid_r, relv,
                bufh0, bufh1, buft0, buft1, outv,
                semh0, semh1, semt0, semt1):
    wid = lax.axis_index("s") * NC + lax.axis_index("c")
    base = wid * BPW
    for b in range(NIDX):
        pltpu.sync_copy(h1.at[pl.ds(base + b * IDXW, IDXW)], oid_h.at[b])
        pltpu.sync_copy(t1.at[pl.ds(base + b * IDXW, IDXW)], oid_t.at[b])
        pltpu.sync_copy(r1.at[pl.ds(base + b * IDXW, IDXW)], oid_r.at[b])
    pltpu.sync_copy(rel2, relv)
    lanes = lax.iota(jnp.int32, L)

    bufh = (bufh0, bufh1)
    buft = (buft0, buft1)
    semh = (semh0, semh1)
    semt = (semt0, semt1)

    def extract(v, j):
        return jnp.sum(jnp.where(lanes == j, v, 0))

    def idx_vecs(bi, ref):
        b = bi // (IDXW // L)
        off = (bi % (IDXW // L)) * L
        return ref[b, pl.ds(off, L)]

    def issue(bi, s):
        vh = idx_vecs(bi, oid_h)
        vt = idx_vecs(bi, oid_t)
        for j in range(L):
            hj = extract(vh, j)
            tj = extract(vt, j)
            pltpu.async_copy(
                ent.at[pl.ds(hj, 1)], bufh[s].at[pl.ds(j, 1)], semh[s])
            pltpu.async_copy(
                ent.at[pl.ds(tj, 1)], buft[s].at[pl.ds(j, 1)], semt[s])

    def drain(s):
        pltpu.make_async_copy(ent.at[pl.ds(0, L)], bufh[s], semh[s]).wait()
        pltpu.make_async_copy(ent.at[pl.ds(0, L)], buft[s], semt[s]).wait()

    def compute(bi, s):
        vr = idx_vecs(bi, oid_r)
        sums = jnp.zeros((L,), jnp.float32)
        for j in range(L):
            rj = extract(vr, j)
            m = lax.shift_right_logical(rj, 1)
            o = jnp.bitwise_and(rj, 1) * D
            acc = None
            for q in range(D // L):
                eh = bufh[s][j, pl.ds(q * L, L)]
                et = buft[s][j, pl.ds(q * L, L)]
                er = relv[m, pl.ds(o + q * L, L)]
                d = (eh - et) + er
                acc = d * d if acc is None else acc + d * d
            tot = jnp.sum(acc)
            sums = jnp.where(lanes == j, tot, sums)
        x = sums + 1e-12
        ib = plsc.bitcast(x, jnp.int32)
        ib = 0x5F3759DF - lax.shift_right_logical(ib, 1)
        y = plsc.bitcast(ib, jnp.float32)
        for _ in range(3):
            y = y * (1.5 - 0.5 * x * y * y)
        outv[pl.ds(bi * L, L)] = -(x * y)

    issue(0, 0)
    issue(1, 1)

    def body(i, carry):
        drain(0)
        compute(2 * i, 0)

        @pl.when(2 * i + 2 < NBATCH)
        def _():
            issue(2 * i + 2, 0)

        drain(1)
        compute(2 * i + 1, 1)

        @pl.when(2 * i + 3 < NBATCH)
        def _():
            issue(2 * i + 3, 1)

        return carry

    lax.fori_loop(0, NBATCH // 2, body, 0)
    pltpu.sync_copy(outv, out.at[pl.ds(base, BPW)])


@jax.jit
def kernel(h, t, r, entity_emb, relation_emb):
    h1 = h.astype(jnp.int32)
    t1 = t.astype(jnp.int32)
    r1 = r.astype(jnp.int32)
    rel2 = relation_emb.reshape(NREL // 2, 2 * D)
    mesh = plsc.VectorSubcoreMesh(
        core_axis_name="c", subcore_axis_name="s",
        num_cores=NC, num_subcores=NS)
    run = pl.kernel(
        _score_body,
        out_type=jax.ShapeDtypeStruct((B,), jnp.float32),
        mesh=mesh,
        compiler_params=pltpu.CompilerParams(needs_layout_passes=False),
        scratch_types=[
            pltpu.VMEM((NIDX, IDXW), jnp.int32),
            pltpu.VMEM((NIDX, IDXW), jnp.int32),
            pltpu.VMEM((NIDX, IDXW), jnp.int32),
            pltpu.VMEM((NREL // 2, 2 * D), jnp.float32),
            pltpu.VMEM((L, D), jnp.float32),
            pltpu.VMEM((L, D), jnp.float32),
            pltpu.VMEM((L, D), jnp.float32),
            pltpu.VMEM((L, D), jnp.float32),
            pltpu.VMEM((BPW,), jnp.float32),
            pltpu.SemaphoreType.DMA,
            pltpu.SemaphoreType.DMA,
            pltpu.SemaphoreType.DMA,
            pltpu.SemaphoreType.DMA,
        ],
    )
    return run(h1, t1, r1, entity_emb, rel2)
